# Initial kernel scaffold; baseline (speedup 1.0000x reference)
import jax
import jax.numpy as jnp
from jax.experimental import pallas as pl


def kernel(positions, hidden_states, ln1_w, wqkv, bqkv, wo, ln2_w, gate_w, gate_bias, expert_wgu, expert_wd, shared_wgu, shared_wd):
    raise NotImplementedError("write your pallas kernel here")


# speedup vs baseline: 1.2903x; 1.2903x over previous
import functools
import jax
import jax.numpy as jnp
from jax.experimental import pallas as pl
from jax.experimental.pallas import tpu as pltpu

T = 2048
H = 2048
NH = 16
NKV = 4
HD = 128
RD = 64
E = 8
TOPK = 2
DFF = 768
SDFF = 768
EPS = 1e-05
THETA = 10000.0

NQKV = (NH + 2 * NKV) * HD
TB = 256
QB = 256
TBM = 512


def _qkv_body(x_ref, w_ref, b_ref, cos_ref, sin_ref, ln_ref, qkv_ref):
    x = x_ref[...]
    inv = jax.lax.rsqrt(jnp.mean(x * x, axis=1, keepdims=True) + EPS)
    h = (x * inv * ln_ref[...]).astype(jnp.bfloat16)
    acc = jnp.dot(h, w_ref[...], preferred_element_type=jnp.float32)
    acc = acc + b_ref[...]
    cos = cos_ref[...]
    sin = sin_ref[...]
    half = RD // 2
    for hh in range(NH + NKV):
        c0 = hh * HD
        x1 = acc[:, c0:c0 + half]
        x2 = acc[:, c0 + half:c0 + RD]
        qkv_ref[:, c0:c0 + half] = (x1 * cos - x2 * sin).astype(jnp.bfloat16)
        qkv_ref[:, c0 + half:c0 + RD] = (x2 * cos + x1 * sin).astype(jnp.bfloat16)
        qkv_ref[:, c0 + RD:c0 + HD] = acc[:, c0 + RD:c0 + HD].astype(jnp.bfloat16)
    v0 = (NH + NKV) * HD
    qkv_ref[:, v0:] = acc[:, v0:].astype(jnp.bfloat16)


def _attn_body(q_ref, k_ref, v_ref, o_ref):
    qi = pl.program_id(1)
    q = q_ref[0]
    s = jax.lax.dot_general(q, k_ref[0], (((1,), (1,)), ((), ())),
                            preferred_element_type=jnp.float32)
    s = s * (HD ** -0.5)
    rows = qi * QB + jax.lax.broadcasted_iota(jnp.int32, (QB, T), 0)
    cols = jax.lax.broadcasted_iota(jnp.int32, (QB, T), 1)
    s = jnp.where(cols <= rows, s, -1e30)
    m = jnp.max(s, axis=1, keepdims=True)
    p = jnp.exp(s - m)
    p = p / jnp.sum(p, axis=1, keepdims=True)
    o_ref[0] = jnp.dot(p.astype(jnp.bfloat16), v_ref[0],
                       preferred_element_type=jnp.float32).astype(jnp.bfloat16)


def _oproj_body(a_ref, w_ref, res_ref, h_ref):
    h_ref[...] = res_ref[...] + jnp.dot(a_ref[...], w_ref[...],
                                        preferred_element_type=jnp.float32)


def _router_body(h_ref, ln_ref, gwt_ref, gb_ref, h2_ref, h2b_ref, cw_ref):
    x = h_ref[...]
    inv = jax.lax.rsqrt(jnp.mean(x * x, axis=1, keepdims=True) + EPS)
    h2 = x * inv * ln_ref[...]
    h2_ref[...] = h2
    h2b_ref[...] = h2.astype(jnp.bfloat16)
    logits = jnp.dot(h2, gwt_ref[...], preferred_element_type=jnp.float32)
    scores = jax.nn.sigmoid(logits)
    choice = scores + gb_ref[...]
    iota = jax.lax.broadcasted_iota(jnp.int32, (TB, E), 1)
    a1 = jnp.argmax(choice, axis=1)
    oh1 = (iota == a1[:, None])
    w1 = jnp.sum(jnp.where(oh1, scores, 0.0), axis=1, keepdims=True)
    choice2 = jnp.where(oh1, -jnp.inf, choice)
    a2 = jnp.argmax(choice2, axis=1)
    oh2 = (iota == a2[:, None])
    w2 = jnp.sum(jnp.where(oh2, scores, 0.0), axis=1, keepdims=True)
    denom = w1 + w2 + 1e-20
    cw_ref[...] = (jnp.where(oh1, w1, 0.0) + jnp.where(oh2, w2, 0.0)) / denom


def _moe_body(h2_ref, wgu_ref, wd_ref, cw_ref, out_ref):
    e = pl.program_id(1)

    @pl.when(e == 0)
    def _():
        out_ref[...] = jnp.zeros_like(out_ref)

    gu = jnp.dot(h2_ref[...], wgu_ref[0], preferred_element_type=jnp.float32)
    g = gu[:, :DFF]
    u = gu[:, DFF:]
    act = (g * jax.nn.sigmoid(g) * u).astype(jnp.bfloat16)
    eo = jnp.dot(act, wd_ref[0], preferred_element_type=jnp.float32)
    out_ref[...] += cw_ref[0] * eo


def _shared_body(h2_ref, wgu_ref, wd_ref, res_ref, moe_ref, out_ref):
    gu = jnp.dot(h2_ref[...], wgu_ref[...], preferred_element_type=jnp.float32)
    g = gu[:, :SDFF]
    u = gu[:, SDFF:]
    act = (g * jax.nn.sigmoid(g) * u).astype(jnp.bfloat16)
    sh = jnp.dot(act, wd_ref[...], preferred_element_type=jnp.float32)
    out_ref[...] = res_ref[...] + moe_ref[...] + sh


def kernel(positions, hidden_states, ln1_w, wqkv, bqkv, wo, ln2_w, gate_w,
           gate_bias, expert_wgu, expert_wd, shared_wgu, shared_wd):
    f32 = jnp.float32
    bf16 = jnp.bfloat16

    inv_freq = 1.0 / (THETA ** (jnp.arange(0, RD, 2).astype(f32) / RD))
    ang = positions.astype(f32)[:, None] * inv_freq[None, :]
    cos = jnp.cos(ang)
    sin = jnp.sin(ang)

    wqkv_b = wqkv.astype(bf16)
    wo_b = wo.astype(bf16)
    wgu_b = expert_wgu.astype(bf16)
    wd_b = expert_wd.astype(bf16)
    swgu_b = shared_wgu.astype(bf16)
    swd_b = shared_wd.astype(bf16)

    qkv = pl.pallas_call(
        _qkv_body,
        grid=(T // TB,),
        in_specs=[
            pl.BlockSpec((TB, H), lambda t: (t, 0)),
            pl.BlockSpec((H, NQKV), lambda t: (0, 0)),
            pl.BlockSpec((1, NQKV), lambda t: (0, 0)),
            pl.BlockSpec((TB, RD // 2), lambda t: (t, 0)),
            pl.BlockSpec((TB, RD // 2), lambda t: (t, 0)),
            pl.BlockSpec((1, H), lambda t: (0, 0)),
        ],
        out_specs=pl.BlockSpec((TB, NQKV), lambda t: (t, 0)),
        out_shape=jax.ShapeDtypeStruct((T, NQKV), bf16),
    )(hidden_states, wqkv_b, bqkv.reshape(1, NQKV), cos, sin,
      ln1_w.reshape(1, H))

    q = qkv[:, :NH * HD].reshape(T, NH, HD).transpose(1, 0, 2)
    k = qkv[:, NH * HD:(NH + NKV) * HD].reshape(T, NKV, HD).transpose(1, 0, 2)
    v = qkv[:, (NH + NKV) * HD:].reshape(T, NKV, HD).transpose(1, 0, 2)

    grp = NH // NKV
    ao = pl.pallas_call(
        _attn_body,
        grid=(NH, T // QB),
        in_specs=[
            pl.BlockSpec((1, QB, HD), lambda h, t: (h, t, 0)),
            pl.BlockSpec((1, T, HD), lambda h, t: (h // grp, 0, 0)),
            pl.BlockSpec((1, T, HD), lambda h, t: (h // grp, 0, 0)),
        ],
        out_specs=pl.BlockSpec((1, QB, HD), lambda h, t: (h, t, 0)),
        out_shape=jax.ShapeDtypeStruct((NH, T, HD), bf16),
    )(q, k, v)
    ao = ao.transpose(1, 0, 2).reshape(T, NH * HD)

    h = pl.pallas_call(
        _oproj_body,
        grid=(T // TB,),
        in_specs=[
            pl.BlockSpec((TB, NH * HD), lambda t: (t, 0)),
            pl.BlockSpec((NH * HD, H), lambda t: (0, 0)),
            pl.BlockSpec((TB, H), lambda t: (t, 0)),
        ],
        out_specs=pl.BlockSpec((TB, H), lambda t: (t, 0)),
        out_shape=jax.ShapeDtypeStruct((T, H), f32),
    )(ao, wo_b, hidden_states)

    h2, h2b, cw = pl.pallas_call(
        _router_body,
        grid=(T // TB,),
        in_specs=[
            pl.BlockSpec((TB, H), lambda t: (t, 0)),
            pl.BlockSpec((1, H), lambda t: (0, 0)),
            pl.BlockSpec((H, E), lambda t: (0, 0)),
            pl.BlockSpec((1, E), lambda t: (0, 0)),
        ],
        out_specs=[
            pl.BlockSpec((TB, H), lambda t: (t, 0)),
            pl.BlockSpec((TB, H), lambda t: (t, 0)),
            pl.BlockSpec((TB, E), lambda t: (t, 0)),
        ],
        out_shape=[
            jax.ShapeDtypeStruct((T, H), f32),
            jax.ShapeDtypeStruct((T, H), bf16),
            jax.ShapeDtypeStruct((T, E), f32),
        ],
    )(h, ln2_w.reshape(1, H), gate_w.T, gate_bias.reshape(1, E))

    cw_col = cw.T.reshape(E, T, 1)
    moe = pl.pallas_call(
        _moe_body,
        grid=(T // TBM, E),
        in_specs=[
            pl.BlockSpec((TBM, H), lambda t, e: (t, 0)),
            pl.BlockSpec((1, H, 2 * DFF), lambda t, e: (e, 0, 0)),
            pl.BlockSpec((1, DFF, H), lambda t, e: (e, 0, 0)),
            pl.BlockSpec((1, TBM, 1), lambda t, e: (e, t, 0)),
        ],
        out_specs=pl.BlockSpec((TBM, H), lambda t, e: (t, 0)),
        out_shape=jax.ShapeDtypeStruct((T, H), f32),
    )(h2b, wgu_b, wd_b, cw_col)

    out = pl.pallas_call(
        _shared_body,
        grid=(T // TB,),
        in_specs=[
            pl.BlockSpec((TB, H), lambda t: (t, 0)),
            pl.BlockSpec((H, 2 * SDFF), lambda t: (0, 0)),
            pl.BlockSpec((SDFF, H), lambda t: (0, 0)),
            pl.BlockSpec((TB, H), lambda t: (t, 0)),
            pl.BlockSpec((TB, H), lambda t: (t, 0)),
        ],
        out_specs=pl.BlockSpec((TB, H), lambda t: (t, 0)),
        out_shape=jax.ShapeDtypeStruct((T, H), f32),
    )(h2b, swgu_b, swd_b, h, moe)

    return out


# speedup vs baseline: 1.4067x; 1.0902x over previous
import functools
import jax
import jax.numpy as jnp
from jax.experimental import pallas as pl
from jax.experimental.pallas import tpu as pltpu

T = 2048
H = 2048
NH = 16
NKV = 4
HD = 128
RD = 64
E = 8
TOPK = 2
DFF = 768
SDFF = 768
EPS = 1e-05
THETA = 10000.0

NQKV = (NH + 2 * NKV) * HD
TB = 256
QB = 256
TBM = 512


def _qkv_body(x_ref, w_ref, b_ref, cos_ref, sin_ref, ln_ref, qkv_ref):
    x = x_ref[...]
    inv = jax.lax.rsqrt(jnp.mean(x * x, axis=1, keepdims=True) + EPS)
    h = (x * inv * ln_ref[...]).astype(jnp.bfloat16)
    acc = jnp.dot(h, w_ref[...], preferred_element_type=jnp.float32)
    acc = acc + b_ref[...]
    cos = cos_ref[...]
    sin = sin_ref[...]
    half = RD // 2
    for hh in range(NH + NKV):
        c0 = hh * HD
        x1 = acc[:, c0:c0 + half]
        x2 = acc[:, c0 + half:c0 + RD]
        qkv_ref[:, c0:c0 + half] = (x1 * cos - x2 * sin).astype(jnp.bfloat16)
        qkv_ref[:, c0 + half:c0 + RD] = (x2 * cos + x1 * sin).astype(jnp.bfloat16)
        qkv_ref[:, c0 + RD:c0 + HD] = acc[:, c0 + RD:c0 + HD].astype(jnp.bfloat16)
    v0 = (NH + NKV) * HD
    qkv_ref[:, v0:] = acc[:, v0:].astype(jnp.bfloat16)


def _attn_body(q_ref, k_ref, v_ref, o_ref):
    qi = pl.program_id(1)
    q = q_ref[...]
    s = jax.lax.dot_general(q, k_ref[...], (((1,), (1,)), ((), ())),
                            preferred_element_type=jnp.float32)
    s = s * (HD ** -0.5)
    rows = qi * QB + jax.lax.broadcasted_iota(jnp.int32, (QB, T), 0)
    cols = jax.lax.broadcasted_iota(jnp.int32, (QB, T), 1)
    s = jnp.where(cols <= rows, s, -1e30)
    m = jnp.max(s, axis=1, keepdims=True)
    p = jnp.exp(s - m)
    p = p / jnp.sum(p, axis=1, keepdims=True)
    o_ref[...] = jnp.dot(p.astype(jnp.bfloat16), v_ref[...],
                         preferred_element_type=jnp.float32).astype(jnp.bfloat16)


def _oproj_body(a_ref, w_ref, res_ref, h_ref):
    h_ref[...] = res_ref[...] + jnp.dot(a_ref[...], w_ref[...],
                                        preferred_element_type=jnp.float32)


def _router_body(h_ref, ln_ref, gwt_ref, gb_ref, h2_ref, h2b_ref, cw_ref):
    x = h_ref[...]
    inv = jax.lax.rsqrt(jnp.mean(x * x, axis=1, keepdims=True) + EPS)
    h2 = x * inv * ln_ref[...]
    h2_ref[...] = h2
    h2b_ref[...] = h2.astype(jnp.bfloat16)
    logits = jnp.dot(h2, gwt_ref[...], preferred_element_type=jnp.float32)
    scores = jax.nn.sigmoid(logits)
    choice = scores + gb_ref[...]
    iota = jax.lax.broadcasted_iota(jnp.int32, (TB, E), 1)
    a1 = jnp.argmax(choice, axis=1)
    oh1 = (iota == a1[:, None])
    w1 = jnp.sum(jnp.where(oh1, scores, 0.0), axis=1, keepdims=True)
    choice2 = jnp.where(oh1, -jnp.inf, choice)
    a2 = jnp.argmax(choice2, axis=1)
    oh2 = (iota == a2[:, None])
    w2 = jnp.sum(jnp.where(oh2, scores, 0.0), axis=1, keepdims=True)
    denom = w1 + w2 + 1e-20
    cw_ref[...] = (jnp.where(oh1, w1, 0.0) + jnp.where(oh2, w2, 0.0)) / denom


def _moe_body(h2_ref, wgu_ref, wd_ref, cw_ref, out_ref):
    e = pl.program_id(1)

    @pl.when(e == 0)
    def _():
        out_ref[...] = jnp.zeros_like(out_ref)

    gu = jnp.dot(h2_ref[...], wgu_ref[0], preferred_element_type=jnp.float32)
    g = gu[:, :DFF]
    u = gu[:, DFF:]
    act = (g * jax.nn.sigmoid(g) * u).astype(jnp.bfloat16)
    eo = jnp.dot(act, wd_ref[0], preferred_element_type=jnp.float32)
    out_ref[...] += cw_ref[0] * eo


def _shared_body(h2_ref, wgu_ref, wd_ref, res_ref, moe_ref, out_ref):
    gu = jnp.dot(h2_ref[...], wgu_ref[...], preferred_element_type=jnp.float32)
    g = gu[:, :SDFF]
    u = gu[:, SDFF:]
    act = (g * jax.nn.sigmoid(g) * u).astype(jnp.bfloat16)
    sh = jnp.dot(act, wd_ref[...], preferred_element_type=jnp.float32)
    out_ref[...] = res_ref[...] + moe_ref[...] + sh


def kernel(positions, hidden_states, ln1_w, wqkv, bqkv, wo, ln2_w, gate_w,
           gate_bias, expert_wgu, expert_wd, shared_wgu, shared_wd):
    f32 = jnp.float32
    bf16 = jnp.bfloat16

    inv_freq = 1.0 / (THETA ** (jnp.arange(0, RD, 2).astype(f32) / RD))
    ang = positions.astype(f32)[:, None] * inv_freq[None, :]
    cos = jnp.cos(ang)
    sin = jnp.sin(ang)

    wqkv_b = wqkv.astype(bf16)
    wo_b = wo.astype(bf16)
    wgu_b = expert_wgu.astype(bf16)
    wd_b = expert_wd.astype(bf16)
    swgu_b = shared_wgu.astype(bf16)
    swd_b = shared_wd.astype(bf16)

    qkv = pl.pallas_call(
        _qkv_body,
        grid=(T // TB,),
        in_specs=[
            pl.BlockSpec((TB, H), lambda t: (t, 0)),
            pl.BlockSpec((H, NQKV), lambda t: (0, 0)),
            pl.BlockSpec((1, NQKV), lambda t: (0, 0)),
            pl.BlockSpec((TB, RD // 2), lambda t: (t, 0)),
            pl.BlockSpec((TB, RD // 2), lambda t: (t, 0)),
            pl.BlockSpec((1, H), lambda t: (0, 0)),
        ],
        out_specs=pl.BlockSpec((TB, NQKV), lambda t: (t, 0)),
        out_shape=jax.ShapeDtypeStruct((T, NQKV), bf16),
    )(hidden_states, wqkv_b, bqkv.reshape(1, NQKV), cos, sin,
      ln1_w.reshape(1, H))

    grp = NH // NKV
    ao = pl.pallas_call(
        _attn_body,
        grid=(NH, T // QB),
        in_specs=[
            pl.BlockSpec((QB, HD), lambda h, t: (t, h)),
            pl.BlockSpec((T, HD), lambda h, t: (0, NH + h // grp)),
            pl.BlockSpec((T, HD), lambda h, t: (0, NH + NKV + h // grp)),
        ],
        out_specs=pl.BlockSpec((QB, HD), lambda h, t: (t, h)),
        out_shape=jax.ShapeDtypeStruct((T, NH * HD), bf16),
    )(qkv, qkv, qkv)

    h = pl.pallas_call(
        _oproj_body,
        grid=(T // TB,),
        in_specs=[
            pl.BlockSpec((TB, NH * HD), lambda t: (t, 0)),
            pl.BlockSpec((NH * HD, H), lambda t: (0, 0)),
            pl.BlockSpec((TB, H), lambda t: (t, 0)),
        ],
        out_specs=pl.BlockSpec((TB, H), lambda t: (t, 0)),
        out_shape=jax.ShapeDtypeStruct((T, H), f32),
    )(ao, wo_b, hidden_states)

    h2, h2b, cw = pl.pallas_call(
        _router_body,
        grid=(T // TB,),
        in_specs=[
            pl.BlockSpec((TB, H), lambda t: (t, 0)),
            pl.BlockSpec((1, H), lambda t: (0, 0)),
            pl.BlockSpec((H, E), lambda t: (0, 0)),
            pl.BlockSpec((1, E), lambda t: (0, 0)),
        ],
        out_specs=[
            pl.BlockSpec((TB, H), lambda t: (t, 0)),
            pl.BlockSpec((TB, H), lambda t: (t, 0)),
            pl.BlockSpec((TB, E), lambda t: (t, 0)),
        ],
        out_shape=[
            jax.ShapeDtypeStruct((T, H), f32),
            jax.ShapeDtypeStruct((T, H), bf16),
            jax.ShapeDtypeStruct((T, E), f32),
        ],
    )(h, ln2_w.reshape(1, H), gate_w.T, gate_bias.reshape(1, E))

    cw_col = cw.T.reshape(E, T, 1)
    moe = pl.pallas_call(
        _moe_body,
        grid=(T // TBM, E),
        in_specs=[
            pl.BlockSpec((TBM, H), lambda t, e: (t, 0)),
            pl.BlockSpec((1, H, 2 * DFF), lambda t, e: (e, 0, 0)),
            pl.BlockSpec((1, DFF, H), lambda t, e: (e, 0, 0)),
            pl.BlockSpec((1, TBM, 1), lambda t, e: (e, t, 0)),
        ],
        out_specs=pl.BlockSpec((TBM, H), lambda t, e: (t, 0)),
        out_shape=jax.ShapeDtypeStruct((T, H), f32),
    )(h2b, wgu_b, wd_b, cw_col)

    out = pl.pallas_call(
        _shared_body,
        grid=(T // TB,),
        in_specs=[
            pl.BlockSpec((TB, H), lambda t: (t, 0)),
            pl.BlockSpec((H, 2 * SDFF), lambda t: (0, 0)),
            pl.BlockSpec((SDFF, H), lambda t: (0, 0)),
            pl.BlockSpec((TB, H), lambda t: (t, 0)),
            pl.BlockSpec((TB, H), lambda t: (t, 0)),
        ],
        out_specs=pl.BlockSpec((TB, H), lambda t: (t, 0)),
        out_shape=jax.ShapeDtypeStruct((T, H), f32),
    )(h2b, swgu_b, swd_b, h, moe)

    return out
